# Initial kernel scaffold; baseline (speedup 1.0000x reference)
import jax
import jax.numpy as jnp
from jax.experimental import pallas as pl


def kernel(x, pos, batch, x_skip, pos_skip, batch_skip, W1, b1, W2, b2):
    raise NotImplementedError("write your pallas kernel here")


# speedup vs baseline: 7.9539x; 7.9539x over previous
import functools

import jax
import jax.numpy as jnp
import numpy as np
from jax import lax
from jax.experimental import pallas as pl
from jax.experimental.pallas import tpu as pltpu
from jax.experimental.pallas import tpu_sc as plsc

N_COARSE = 4096
N_FINE = 16384
D_IN = 256
D_SKIP = 128
D_HID = 256
D_OUT = 256
K = 3

BM = 512


def _topk_body(a_ref, b_ref, ysq_ref, i0_ref, i1_ref, i2_ref,
               w0_ref, w1_ref, w2_ref):
    a = a_ref[...]
    b = b_ref[...]
    d2r = jnp.dot(a, b, preferred_element_type=jnp.float32,
                  precision=lax.Precision.HIGHEST)
    n = d2r.shape[1]
    idxrow = lax.broadcasted_iota(jnp.int32, d2r.shape, 1)
    big_i = jnp.int32(n)
    inf = jnp.float32(np.inf)

    mins, idxs = [], []
    d = d2r
    for _ in range(K):
        mk = jnp.min(d, axis=1, keepdims=True)
        cand = jnp.where(d == mk, idxrow, big_i)
        ik = jnp.min(cand, axis=1, keepdims=True)
        d = jnp.where(cand == ik, inf, d)
        mins.append(mk)
        idxs.append(ik)

    ysq = ysq_ref[...]
    ws = [1.0 / jnp.maximum(mk + ysq, 1e-16) for mk in mins]
    den = ws[0] + ws[1] + ws[2]
    i0_ref[...] = idxs[0]
    i1_ref[...] = idxs[1]
    i2_ref[...] = idxs[2]
    w0_ref[...] = ws[0] / den
    w1_ref[...] = ws[1] / den
    w2_ref[...] = ws[2] / den


def _topk_call(a, b, ysq):
    m = a.shape[0]
    grid = (m // BM,)
    col = pl.BlockSpec((BM, 1), lambda i: (i, 0))
    return pl.pallas_call(
        _topk_body,
        grid=grid,
        in_specs=[
            pl.BlockSpec((BM, 4), lambda i: (i, 0)),
            pl.BlockSpec((4, N_COARSE), lambda i: (0, 0)),
            col,
        ],
        out_specs=[col] * 6,
        out_shape=[jax.ShapeDtypeStruct((m, 1), jnp.int32)] * 3
        + [jax.ShapeDtypeStruct((m, 1), jnp.float32)] * 3,
    )(a, b, ysq)


_NC = 2
_NS = 16
_NW = _NC * _NS
_GATHER_B = K * N_FINE
_B_PER_W = _GATHER_B // _NW
_CHUNK = 256
_N_CHUNKS = _B_PER_W // _CHUNK


def _sc_gather_body(table_hbm, idx_hbm, out_hbm, idx_v, rows_v, sem):
    wid = lax.axis_index("s") * _NC + lax.axis_index("c")
    base = wid * _B_PER_W
    for ci in range(_N_CHUNKS):
        off = base + ci * _CHUNK
        pltpu.sync_copy(idx_hbm.at[pl.ds(off, _CHUNK)], idx_v)
        pltpu.async_copy(table_hbm.at[idx_v], rows_v, sem).wait()
        pltpu.sync_copy(rows_v, out_hbm.at[pl.ds(off, _CHUNK)])


@functools.cache
def _sc_gather():
    return functools.partial(
        pl.kernel,
        mesh=plsc.VectorSubcoreMesh(core_axis_name="c", subcore_axis_name="s"),
        out_type=jax.ShapeDtypeStruct((_GATHER_B, D_IN), jnp.float32),
        scratch_types=[
            pltpu.VMEM((_CHUNK,), jnp.int32),
            pltpu.VMEM((_CHUNK, D_IN), jnp.float32),
            pltpu.SemaphoreType.DMA,
        ],
    )(_sc_gather_body)


def _mlp_body(g0_ref, g1_ref, g2_ref, w0_ref, w1_ref, w2_ref, xs_ref,
              w1a_ref, w1b_ref, b1_ref, w2m_ref, b2_ref, o_ref):
    xi = (w0_ref[...] * g0_ref[...]
          + w1_ref[...] * g1_ref[...]
          + w2_ref[...] * g2_ref[...])
    h = jnp.dot(xi, w1a_ref[...], preferred_element_type=jnp.float32,
                precision=lax.Precision.HIGHEST)
    h = h + jnp.dot(xs_ref[...], w1b_ref[...],
                    preferred_element_type=jnp.float32,
                    precision=lax.Precision.HIGHEST)
    h = jnp.maximum(h + b1_ref[...], 0.0)
    o = jnp.dot(h, w2m_ref[...], preferred_element_type=jnp.float32,
                precision=lax.Precision.HIGHEST)
    o_ref[...] = jnp.maximum(o + b2_ref[...], 0.0)


def _mlp_call(g0, g1, g2, w0, w1, w2, x_skip, w1a, w1b, b1, w2m, b2):
    m = g0.shape[0]
    grid = (m // BM,)
    row = pl.BlockSpec((BM, D_IN), lambda i: (i, 0))
    col = pl.BlockSpec((BM, 1), lambda i: (i, 0))
    full = lambda r, c: pl.BlockSpec((r, c), lambda i: (0, 0))
    return pl.pallas_call(
        _mlp_body,
        grid=grid,
        in_specs=[
            row, row, row, col, col, col,
            pl.BlockSpec((BM, D_SKIP), lambda i: (i, 0)),
            full(D_IN, D_HID), full(D_SKIP, D_HID), full(1, D_HID),
            full(D_HID, D_OUT), full(1, D_OUT),
        ],
        out_specs=pl.BlockSpec((BM, D_OUT), lambda i: (i, 0)),
        out_shape=jax.ShapeDtypeStruct((m, D_OUT), jnp.float32),
    )(g0, g1, g2, w0, w1, w2, x_skip, w1a, w1b, b1, w2m, b2)


def kernel(x, pos, batch, x_skip, pos_skip, batch_skip, W1, b1, W2, b2):
    m = pos_skip.shape[0]
    a = jnp.concatenate([pos_skip, jnp.ones((m, 1), jnp.float32)], axis=1)
    bmat = jnp.concatenate([-2.0 * pos.T,
                            jnp.sum(pos * pos, axis=1)[None, :]], axis=0)
    ysq = jnp.sum(pos_skip * pos_skip, axis=1, keepdims=True)

    i0, i1, i2, w0, w1, w2 = _topk_call(a, bmat, ysq)

    idx_flat = jnp.concatenate([i0, i1, i2], axis=0).reshape(-1)
    gathered = _sc_gather()(x, idx_flat)
    g0, g1, g2 = gathered[:m], gathered[m:2 * m], gathered[2 * m:]

    h = _mlp_call(g0, g1, g2, w0, w1, w2, x_skip,
                  W1[:D_IN], W1[D_IN:], b1[None, :], W2, b2[None, :])
    return (h, pos_skip, batch_skip)


# speedup vs baseline: 12.2897x; 1.5451x over previous
import functools

import jax
import jax.numpy as jnp
import numpy as np
from jax import lax
from jax.experimental import pallas as pl
from jax.experimental.pallas import tpu as pltpu
from jax.experimental.pallas import tpu_sc as plsc

N_COARSE = 4096
N_FINE = 16384
D_IN = 256
D_SKIP = 128
D_HID = 256
D_OUT = 256
K = 3

BM = 1024
BMC = 512


def _topk_body(a_ref, b_ref, ysq_ref, i0_ref, i1_ref, i2_ref,
               w0_ref, w1_ref, w2_ref):
    a = a_ref[...]
    b = b_ref[...]
    d2r = jnp.dot(a, b, preferred_element_type=jnp.float32)
    n = d2r.shape[1]
    idxrow = lax.broadcasted_iota(jnp.int32, d2r.shape, 1).astype(jnp.float32)
    big_f = jnp.float32(n)
    inf = jnp.float32(np.inf)

    mins, idxs = [], []
    d = d2r
    for _ in range(K):
        mk = jnp.min(d, axis=1, keepdims=True)
        cand = jnp.where(d == mk, idxrow, big_f)
        ik = jnp.min(cand, axis=1, keepdims=True)
        d = jnp.where(cand == ik, inf, d)
        mins.append(mk)
        idxs.append(ik)

    ysq = ysq_ref[...]
    ws = [1.0 / jnp.maximum(mk + ysq, 1e-16) for mk in mins]
    den = ws[0] + ws[1] + ws[2]
    i0_ref[...] = idxs[0].astype(jnp.int32)
    i1_ref[...] = idxs[1].astype(jnp.int32)
    i2_ref[...] = idxs[2].astype(jnp.int32)
    w0_ref[...] = ws[0] / den
    w1_ref[...] = ws[1] / den
    w2_ref[...] = ws[2] / den


def _topk_call(a, b, ysq):
    m = a.shape[0]
    grid = (m // BM,)
    col = pl.BlockSpec((BM, 1), lambda i: (i, 0))
    return pl.pallas_call(
        _topk_body,
        grid=grid,
        in_specs=[
            pl.BlockSpec((BM, 4), lambda i: (i, 0)),
            pl.BlockSpec((4, N_COARSE), lambda i: (0, 0)),
            col,
        ],
        out_specs=[col] * 6,
        out_shape=[jax.ShapeDtypeStruct((m, 1), jnp.int32)] * 3
        + [jax.ShapeDtypeStruct((m, 1), jnp.float32)] * 3,
    )(a, b, ysq)


_NC = 2
_NS = 16
_NW = _NC * _NS
_GATHER_B = K * N_FINE
_B_PER_W = _GATHER_B // _NW
_CHUNK = 256
_N_CHUNKS = _B_PER_W // _CHUNK


def _sc_gather_body(table_hbm, idx_hbm, out_hbm, idx_v, rows_v, sem):
    wid = lax.axis_index("s") * _NC + lax.axis_index("c")
    base = wid * _B_PER_W
    for ci in range(_N_CHUNKS):
        off = base + ci * _CHUNK
        pltpu.sync_copy(idx_hbm.at[pl.ds(off, _CHUNK)], idx_v)
        pltpu.async_copy(table_hbm.at[idx_v], rows_v, sem).wait()
        pltpu.sync_copy(rows_v, out_hbm.at[pl.ds(off, _CHUNK)])


@functools.cache
def _sc_gather():
    return functools.partial(
        pl.kernel,
        mesh=plsc.VectorSubcoreMesh(core_axis_name="c", subcore_axis_name="s"),
        out_type=jax.ShapeDtypeStruct((_GATHER_B, D_IN), jnp.float32),
        scratch_types=[
            pltpu.VMEM((_CHUNK,), jnp.int32),
            pltpu.VMEM((_CHUNK, D_IN), jnp.float32),
            pltpu.SemaphoreType.DMA,
        ],
    )(_sc_gather_body)


def _mlp_body(g0_ref, g1_ref, g2_ref, w0_ref, w1_ref, w2_ref, xs_ref,
              w1a_ref, w1b_ref, b1_ref, w2m_ref, b2_ref, o_ref):
    xi = (w0_ref[...] * g0_ref[...]
          + w1_ref[...] * g1_ref[...]
          + w2_ref[...] * g2_ref[...])
    h = jnp.dot(xi, w1a_ref[...], preferred_element_type=jnp.float32,
                precision=lax.Precision.HIGHEST)
    h = h + jnp.dot(xs_ref[...], w1b_ref[...],
                    preferred_element_type=jnp.float32,
                    precision=lax.Precision.HIGHEST)
    h = jnp.maximum(h + b1_ref[...], 0.0)
    o = jnp.dot(h, w2m_ref[...], preferred_element_type=jnp.float32,
                precision=lax.Precision.HIGHEST)
    o_ref[...] = jnp.maximum(o + b2_ref[...], 0.0)


def _mlp_call(g0, g1, g2, w0, w1, w2, x_skip, w1a, w1b, b1, w2m, b2):
    m = g0.shape[0]
    grid = (m // BMC,)
    row = pl.BlockSpec((BMC, D_IN), lambda i: (i, 0))
    col = pl.BlockSpec((BMC, 1), lambda i: (i, 0))
    full = lambda r, c: pl.BlockSpec((r, c), lambda i: (0, 0))
    return pl.pallas_call(
        _mlp_body,
        grid=grid,
        in_specs=[
            row, row, row, col, col, col,
            pl.BlockSpec((BMC, D_SKIP), lambda i: (i, 0)),
            full(D_IN, D_HID), full(D_SKIP, D_HID), full(1, D_HID),
            full(D_HID, D_OUT), full(1, D_OUT),
        ],
        out_specs=pl.BlockSpec((BMC, D_OUT), lambda i: (i, 0)),
        out_shape=jax.ShapeDtypeStruct((m, D_OUT), jnp.float32),
    )(g0, g1, g2, w0, w1, w2, x_skip, w1a, w1b, b1, w2m, b2)


def kernel(x, pos, batch, x_skip, pos_skip, batch_skip, W1, b1, W2, b2):
    m = pos_skip.shape[0]
    a = jnp.concatenate([pos_skip, jnp.ones((m, 1), jnp.float32)], axis=1)
    bmat = jnp.concatenate([-2.0 * pos.T,
                            jnp.sum(pos * pos, axis=1)[None, :]], axis=0)
    ysq = jnp.sum(pos_skip * pos_skip, axis=1, keepdims=True)

    i0, i1, i2, w0, w1, w2 = _topk_call(a, bmat, ysq)

    idx_flat = jnp.concatenate([i0, i1, i2], axis=0).reshape(-1)
    gathered = _sc_gather()(x, idx_flat)
    g0, g1, g2 = gathered[:m], gathered[m:2 * m], gathered[2 * m:]

    h = _mlp_call(g0, g1, g2, w0, w1, w2, x_skip,
                  W1[:D_IN], W1[D_IN:], b1[None, :], W2, b2[None, :])
    return (h, pos_skip, batch_skip)
